# baseline (speedup 1.0000x reference)
import functools

import jax
import jax.numpy as jnp
from jax import lax
from jax.experimental import pallas as pl
from jax.experimental.pallas import tpu as pltpu
from jax.experimental.pallas import tpu_sc as plsc

_NC = 2
_NS = 16
_NW = _NC * _NS
_L = 16

_B = 16384
_V = 100000
_D = 32

_NBLK = (_V + 127) // 128
_BPT = (_NBLK + _NW - 1) // _NW
_CHUNK = 1024
_JPT = _B // _NS
_SLAB = 64
_RING = 4


def kernel(x, table):
    mesh = plsc.VectorSubcoreMesh(core_axis_name="c", subcore_axis_name="s")

    @functools.partial(
        pl.kernel,
        mesh=mesh,
        out_type=(
            jax.ShapeDtypeStruct((_D, _B), jnp.float32),
            jax.ShapeDtypeStruct((_D, _B), jnp.float32),
        ),
        scratch_types=[
            pltpu.VMEM((_BPT, _D, 128), jnp.float32),
            pltpu.VMEM((_CHUNK,), jnp.int32),
            pltpu.VMEM((_CHUNK,), jnp.int32),
            pltpu.VMEM((_RING, _L, 128), jnp.float32),
            pltpu.VMEM((_SLAB, 128), jnp.float32),
            pltpu.VMEM((_D, 128), jnp.float32),
            pltpu.HBM((_B, 128), jnp.float32),
            pltpu.HBM((_B, 128), jnp.float32),
            pltpu.SemaphoreType.DMA,
            [pltpu.SemaphoreType.DMA] * _RING,
            pltpu.SemaphoreType.DMA,
        ],
        compiler_params=pltpu.CompilerParams(
            use_tc_tiling_on_sc=True, needs_layout_passes=False
        ),
    )
    def _emb(x_hbm, tableT_hbm, out0_hbm, out1_hbm, blocks_v, idx_v, wl_v,
             grp_v, slab_v, asm_v, sp0, sp1, sem_t, sems_r, sem_m):
        cid = lax.axis_index("c")
        sid = lax.axis_index("s")
        wid = sid * _NC + cid
        zeros16 = jnp.zeros((_L,), jnp.float32)
        zidx16 = jnp.zeros((_L,), jnp.int32)

        def on_my_sc(fn):
            @pl.when(cid == 0)
            def _():
                fn(sp0)
            @pl.when(cid == 1)
            def _():
                fn(sp1)

        def _stream(b):
            g = wid + _NW * b
            @pl.when(g < _NBLK)
            def _():
                pltpu.async_copy(
                    tableT_hbm.at[:, pl.ds(g * 128, 128)], blocks_v.at[b],
                    sem_t)

        def _stream_wait(b):
            g = wid + _NW * b
            @pl.when(g < _NBLK)
            def _():
                pltpu.make_async_copy(
                    tableT_hbm.at[:, pl.ds(0, 128)], blocks_v.at[b],
                    sem_t).wait()

        for b in range(min(8, _BPT)):
            _stream(b)
        for b in range(8, _BPT):
            _stream_wait(b - 8)
            _stream(b)

        def _zero_body(r, _):
            slab_v[r, pl.ds(0, _L)] = zeros16
            slab_v[r, pl.ds(_L, _L)] = zeros16
            return 0
        lax.fori_loop(0, _SLAB, _zero_body, 0)

        n_spray = _JPT // _SLAB
        for i in range(n_spray):
            on_my_sc(lambda sp, i=i: pltpu.async_copy(
                slab_v.at[:, pl.ds(0, _D)],
                sp.at[pl.ds(sid * _JPT + i * _SLAB, _SLAB), pl.ds(0, _D)],
                sem_m))
        for i in range(n_spray):
            on_my_sc(lambda sp: pltpu.make_async_copy(
                slab_v.at[:, pl.ds(0, _D)],
                sp.at[pl.ds(0, _SLAB), pl.ds(0, _D)], sem_m).wait())

        for b in range(max(_BPT - 8, 0), _BPT):
            _stream_wait(b)

        plsc.subcore_barrier()

        def _chunk_body(ci, carry):
            base = ci * _CHUNK
            pltpu.sync_copy(x_hbm.at[pl.ds(base, _CHUNK)], idx_v)

            def _scan_body(k, n):
                iota = lax.iota(jnp.int32, _L)
                idx = idx_v[pl.ds(k * _L, _L)]
                mine = ((idx >> 7) & (_NW - 1)) == wid
                jpos = base + k * _L + iota
                csum = plsc.cumsum(mine.astype(jnp.int32))
                plsc.store_scatter(wl_v, [n + csum - 1], jpos, mask=mine)
                return n + csum[_L - 1]

            n_mine = lax.fori_loop(0, _CHUNK // _L, _scan_body, 0)
            n_grp = (n_mine + _L - 1) // _L

            def _ext4_body(kk, _):
                for s in range(_RING):
                    gi = kk * _RING + s

                    @pl.when((gi >= _RING) & (gi - _RING < n_grp))
                    def _(s=s):
                        on_my_sc(lambda sp: pltpu.make_async_copy(
                            grp_v.at[s], sp.at[zidx16], sems_r[s]).wait())

                    @pl.when(gi < n_grp)
                    def _(s=s, gi=gi):
                        iota = lax.iota(jnp.int32, _L)
                        jv_raw = wl_v[pl.ds(gi * _L, _L)]
                        valid = (gi * _L + iota) < n_mine
                        jv0 = jnp.broadcast_to(jv_raw[0], (_L,))
                        jv = jnp.where(valid, jv_raw, jv0)
                        idxs = plsc.load_gather(idx_v, [jv - base])
                        bslot = idxs >> 12
                        lane = idxs & 127
                        sv = jnp.full((_L,), s, jnp.int32)
                        for c in range(_D):
                            vals = plsc.load_gather(
                                blocks_v,
                                [bslot, jnp.full((_L,), c, jnp.int32), lane])
                            plsc.store_scatter(
                                grp_v,
                                [sv, iota, jnp.full((_L,), c, jnp.int32)],
                                vals)
                        on_my_sc(lambda sp: pltpu.async_copy(
                            grp_v.at[s], sp.at[jv], sems_r[s]))
                return 0

            n_kk = (n_grp + _RING - 1) // _RING
            lax.fori_loop(0, n_kk, _ext4_body, 0)

            for s in range(_RING):
                @pl.when(s < n_grp)
                def _(s=s):
                    on_my_sc(lambda sp: pltpu.make_async_copy(
                        grp_v.at[s], sp.at[zidx16], sems_r[s]).wait())
            return carry

        lax.fori_loop(0, _B // _CHUNK, _chunk_body, 0)

        plsc.subcore_barrier()

        def _write_block(blk, _):
            for half in range(128 // _SLAB):
                j0 = sid * _JPT + blk * 128 + half * _SLAB
                on_my_sc(lambda sp: pltpu.sync_copy(
                    sp.at[pl.ds(j0, _SLAB), pl.ds(0, _D)],
                    slab_v.at[:, pl.ds(0, _D)]))

                for c in range(_D):
                    for q in range(_SLAB // _L):
                        iota = lax.iota(jnp.int32, _L)
                        v = plsc.load_gather(
                            slab_v,
                            [q * _L + iota, jnp.full((_L,), c, jnp.int32)])
                        asm_v[c, pl.ds(half * _SLAB + q * _L, _L)] = v

            jout = sid * _JPT + blk * 128
            @pl.when(cid == 0)
            def _():
                pltpu.sync_copy(asm_v, out0_hbm.at[:, pl.ds(jout, 128)])
            @pl.when(cid == 1)
            def _():
                pltpu.sync_copy(asm_v, out1_hbm.at[:, pl.ds(jout, 128)])
            return 0

        lax.fori_loop(0, _JPT // 128, _write_block, 0)

    o0, o1 = _emb(x, table.T)
    return (o0 + o1).T


# speedup vs baseline: 1.1017x; 1.1017x over previous
import functools

import jax
import jax.numpy as jnp
from jax import lax
from jax.experimental import pallas as pl
from jax.experimental.pallas import tpu as pltpu
from jax.experimental.pallas import tpu_sc as plsc

_NC = 2
_NS = 16
_NW = _NC * _NS
_L = 16

_B = 16384
_V = 100000
_D = 32

_NBLK = (_V + 127) // 128
_BPT = (_NBLK + _NW - 1) // _NW
_CHUNK = 1024
_JPT = _B // _NS
_SLAB = 64
_RING = 4


def kernel(x, table):
    mesh = plsc.VectorSubcoreMesh(core_axis_name="c", subcore_axis_name="s")

    @functools.partial(
        pl.kernel,
        mesh=mesh,
        out_type=(
            jax.ShapeDtypeStruct((_D, _B), jnp.float32),
            jax.ShapeDtypeStruct((_D, _B), jnp.float32),
        ),
        scratch_types=[
            pltpu.VMEM((_BPT, _D, 128), jnp.float32),
            pltpu.VMEM((2, _CHUNK), jnp.int32),
            pltpu.VMEM((_CHUNK,), jnp.int32),
            pltpu.VMEM((_RING, _L, 128), jnp.float32),
            pltpu.VMEM((_SLAB, 128), jnp.float32),
            pltpu.VMEM((_D, 128), jnp.float32),
            pltpu.HBM((_B, 128), jnp.float32),
            pltpu.HBM((_B, 128), jnp.float32),
            pltpu.SemaphoreType.DMA,
            [pltpu.SemaphoreType.DMA] * _RING,
            pltpu.SemaphoreType.DMA,
        ],
        compiler_params=pltpu.CompilerParams(
            use_tc_tiling_on_sc=True, needs_layout_passes=False
        ),
    )
    def _emb(x_hbm, tableT_hbm, out0_hbm, out1_hbm, blocks_v, idx_v, wl_v,
             grp_v, slab_v, asm_v, sp0, sp1, sem_t, sems_r, sem_m):
        cid = lax.axis_index("c")
        sid = lax.axis_index("s")
        wid = sid * _NC + cid
        zeros16 = jnp.zeros((_L,), jnp.float32)
        zidx16 = jnp.zeros((_L,), jnp.int32)

        def on_my_sc(fn):
            @pl.when(cid == 0)
            def _():
                fn(sp0)
            @pl.when(cid == 1)
            def _():
                fn(sp1)

        def _stream(b):
            g = wid + _NW * b
            @pl.when(g < _NBLK)
            def _():
                pltpu.async_copy(
                    tableT_hbm.at[:, pl.ds(g * 128, 128)], blocks_v.at[b],
                    sem_t)

        def _stream_wait(b):
            g = wid + _NW * b
            @pl.when(g < _NBLK)
            def _():
                pltpu.make_async_copy(
                    tableT_hbm.at[:, pl.ds(0, 128)], blocks_v.at[b],
                    sem_t).wait()

        for b in range(min(8, _BPT)):
            _stream(b)
        for b in range(8, _BPT):
            _stream_wait(b - 8)
            _stream(b)

        def _zero_body(r, _):
            slab_v[r, pl.ds(0, _L)] = zeros16
            slab_v[r, pl.ds(_L, _L)] = zeros16
            return 0
        lax.fori_loop(0, _SLAB, _zero_body, 0)

        n_spray = _JPT // _SLAB
        for i in range(n_spray):
            on_my_sc(lambda sp, i=i: pltpu.async_copy(
                slab_v.at[:, pl.ds(0, _D)],
                sp.at[pl.ds(sid * _JPT + i * _SLAB, _SLAB), pl.ds(0, _D)],
                sem_m))
        for i in range(n_spray):
            on_my_sc(lambda sp: pltpu.make_async_copy(
                slab_v.at[:, pl.ds(0, _D)],
                sp.at[pl.ds(0, _SLAB), pl.ds(0, _D)], sem_m).wait())

        for b in range(max(_BPT - 8, 0), _BPT):
            _stream_wait(b)

        plsc.subcore_barrier()

        pltpu.async_copy(x_hbm.at[pl.ds(0, _CHUNK)], idx_v.at[0], sem_m)

        def _chunk_body(ci, carry):
            base = ci * _CHUNK
            buf = ci & 1
            pltpu.make_async_copy(
                x_hbm.at[pl.ds(0, _CHUNK)], idx_v.at[0], sem_m).wait()
            @pl.when(ci + 1 < _B // _CHUNK)
            def _():
                pltpu.async_copy(
                    x_hbm.at[pl.ds(base + _CHUNK, _CHUNK)],
                    idx_v.at[1 - buf], sem_m)

            def _scan_body(k4, n):
                iota = lax.iota(jnp.int32, _L)
                for u in range(4):
                    k = k4 * 4 + u
                    idx = idx_v[buf, pl.ds(k * _L, _L)]
                    mine = ((idx >> 7) & (_NW - 1)) == wid
                    jpos = base + k * _L + iota
                    csum = plsc.cumsum(mine.astype(jnp.int32))
                    plsc.store_scatter(
                        wl_v, [n + csum - 1], jpos, mask=mine)
                    n = n + csum[_L - 1]
                return n

            n_mine = lax.fori_loop(0, _CHUNK // _L // 4, _scan_body, 0)
            n_grp = (n_mine + _L - 1) // _L

            def _ext4_body(kk, _):
                for s in range(_RING):
                    gi = kk * _RING + s

                    @pl.when((gi >= _RING) & (gi - _RING < n_grp))
                    def _(s=s):
                        on_my_sc(lambda sp: pltpu.make_async_copy(
                            grp_v.at[s], sp.at[zidx16], sems_r[s]).wait())

                    @pl.when(gi < n_grp)
                    def _(s=s, gi=gi):
                        iota = lax.iota(jnp.int32, _L)
                        jv_raw = wl_v[pl.ds(gi * _L, _L)]
                        valid = (gi * _L + iota) < n_mine
                        jv0 = jnp.broadcast_to(jv_raw[0], (_L,))
                        jv = jnp.where(valid, jv_raw, jv0)
                        idxs = plsc.load_gather(
                            idx_v, [jnp.full((_L,), buf, jnp.int32),
                                    jv - base])
                        bslot = idxs >> 12
                        lane = idxs & 127
                        sv = jnp.full((_L,), s, jnp.int32)
                        for c in range(_D):
                            vals = plsc.load_gather(
                                blocks_v,
                                [bslot, jnp.full((_L,), c, jnp.int32), lane])
                            plsc.store_scatter(
                                grp_v,
                                [sv, iota, jnp.full((_L,), c, jnp.int32)],
                                vals)
                        on_my_sc(lambda sp: pltpu.async_copy(
                            grp_v.at[s], sp.at[jv], sems_r[s]))
                return 0

            n_kk = (n_grp + _RING - 1) // _RING
            lax.fori_loop(0, n_kk, _ext4_body, 0)

            for s in range(_RING):
                @pl.when(s < n_grp)
                def _(s=s):
                    on_my_sc(lambda sp: pltpu.make_async_copy(
                        grp_v.at[s], sp.at[zidx16], sems_r[s]).wait())
            return carry

        lax.fori_loop(0, _B // _CHUNK, _chunk_body, 0)

        plsc.subcore_barrier()

        def _write_block(blk, _):
            for half in range(128 // _SLAB):
                j0 = sid * _JPT + blk * 128 + half * _SLAB
                on_my_sc(lambda sp: pltpu.sync_copy(
                    sp.at[pl.ds(j0, _SLAB), pl.ds(0, _D)],
                    slab_v.at[:, pl.ds(0, _D)]))

                for c in range(_D):
                    for q in range(_SLAB // _L):
                        iota = lax.iota(jnp.int32, _L)
                        v = plsc.load_gather(
                            slab_v,
                            [q * _L + iota, jnp.full((_L,), c, jnp.int32)])
                        asm_v[c, pl.ds(half * _SLAB + q * _L, _L)] = v

            jout = sid * _JPT + blk * 128
            @pl.when(cid == 0)
            def _():
                pltpu.sync_copy(asm_v, out0_hbm.at[:, pl.ds(jout, 128)])
            @pl.when(cid == 1)
            def _():
                pltpu.sync_copy(asm_v, out1_hbm.at[:, pl.ds(jout, 128)])
            return 0

        lax.fori_loop(0, _JPT // 128, _write_block, 0)

    o0, o1 = _emb(x, table.T)
    return (o0 + o1).T


# speedup vs baseline: 1.1456x; 1.0399x over previous
import functools

import jax
import jax.numpy as jnp
from jax import lax
from jax.experimental import pallas as pl
from jax.experimental.pallas import tpu as pltpu
from jax.experimental.pallas import tpu_sc as plsc

_NC = 2
_NS = 16
_NW = _NC * _NS
_L = 16

_B = 16384
_V = 100000
_D = 32

_NBLK = (_V + 127) // 128
_BPT = (_NBLK + _NW - 1) // _NW
_CHUNK = 1024
_JPT = _B // _NS
_SLAB = 64
_RING = 4


def kernel(x, table):
    mesh = plsc.VectorSubcoreMesh(core_axis_name="c", subcore_axis_name="s")

    @functools.partial(
        pl.kernel,
        mesh=mesh,
        out_type=(
            jax.ShapeDtypeStruct((_D, _B), jnp.float32),
            jax.ShapeDtypeStruct((_D, _B), jnp.float32),
        ),
        scratch_types=[
            pltpu.VMEM((_BPT, _D, 128), jnp.float32),
            pltpu.VMEM((2, _CHUNK), jnp.int32),
            pltpu.VMEM((_CHUNK,), jnp.int32),
            pltpu.VMEM((_RING, _L, 128), jnp.float32),
            pltpu.VMEM((_SLAB, 128), jnp.float32),
            pltpu.VMEM((_D, 128), jnp.float32),
            pltpu.HBM((_B, 128), jnp.float32),
            pltpu.HBM((_B, 128), jnp.float32),
            pltpu.SemaphoreType.DMA,
            [pltpu.SemaphoreType.DMA] * _RING,
            pltpu.SemaphoreType.DMA,
        ],
        compiler_params=pltpu.CompilerParams(
            use_tc_tiling_on_sc=True, needs_layout_passes=False
        ),
    )
    def _emb(x_hbm, tableT_hbm, out0_hbm, out1_hbm, blocks_v, idx_v, wl_v,
             grp_v, slab_v, asm_v, sp0, sp1, sem_t, sems_r, sem_m):
        cid = lax.axis_index("c")
        sid = lax.axis_index("s")
        wid = sid * _NC + cid
        zeros16 = jnp.zeros((_L,), jnp.float32)
        zidx16 = jnp.zeros((_L,), jnp.int32)

        def on_my_sc(fn):
            @pl.when(cid == 0)
            def _():
                fn(sp0)
            @pl.when(cid == 1)
            def _():
                fn(sp1)

        def _stream(b):
            g = wid + _NW * b
            @pl.when(g < _NBLK)
            def _():
                pltpu.async_copy(
                    tableT_hbm.at[:, pl.ds(g * 128, 128)], blocks_v.at[b],
                    sem_t)

        def _stream_wait(b):
            g = wid + _NW * b
            @pl.when(g < _NBLK)
            def _():
                pltpu.make_async_copy(
                    tableT_hbm.at[:, pl.ds(0, 128)], blocks_v.at[b],
                    sem_t).wait()

        for b in range(min(8, _BPT)):
            _stream(b)
        for b in range(8, _BPT):
            _stream_wait(b - 8)
            _stream(b)

        def _zero_body(r, _):
            slab_v[r, pl.ds(0, _L)] = zeros16
            slab_v[r, pl.ds(_L, _L)] = zeros16
            return 0
        lax.fori_loop(0, _SLAB, _zero_body, 0)

        n_spray = _JPT // _SLAB
        for i in range(n_spray):
            on_my_sc(lambda sp, i=i: pltpu.async_copy(
                slab_v.at[:, pl.ds(0, _D)],
                sp.at[pl.ds(sid * _JPT + i * _SLAB, _SLAB), pl.ds(0, _D)],
                sem_m))
        for i in range(n_spray):
            on_my_sc(lambda sp: pltpu.make_async_copy(
                slab_v.at[:, pl.ds(0, _D)],
                sp.at[pl.ds(0, _SLAB), pl.ds(0, _D)], sem_m).wait())

        for b in range(max(_BPT - 8, 0), _BPT):
            _stream_wait(b)

        plsc.subcore_barrier()

        pltpu.async_copy(x_hbm.at[pl.ds(0, _CHUNK)], idx_v.at[0], sem_m)

        def _chunk_body(ci, carry):
            base = ci * _CHUNK
            buf = ci & 1
            pltpu.make_async_copy(
                x_hbm.at[pl.ds(0, _CHUNK)], idx_v.at[0], sem_m).wait()
            @pl.when(ci + 1 < _B // _CHUNK)
            def _():
                pltpu.async_copy(
                    x_hbm.at[pl.ds(base + _CHUNK, _CHUNK)],
                    idx_v.at[1 - buf], sem_m)

            def _scan_body(k4, n):
                iota = lax.iota(jnp.int32, _L)
                mines, csums = [], []
                for u in range(4):
                    k = k4 * 4 + u
                    idx = idx_v[buf, pl.ds(k * _L, _L)]
                    mines.append(((idx >> 7) & (_NW - 1)) == wid)
                    csums.append(plsc.cumsum(mines[u].astype(jnp.int32)))
                offs = n
                for u in range(4):
                    k = k4 * 4 + u
                    jpos = base + k * _L + iota
                    plsc.store_scatter(
                        wl_v, [offs + csums[u] - 1], jpos, mask=mines[u])
                    offs = offs + csums[u][_L - 1]
                return offs

            n_mine = lax.fori_loop(0, _CHUNK // _L // 4, _scan_body, 0)
            n_grp = (n_mine + _L - 1) // _L

            def _ext4_body(kk, _):
                for s in range(_RING):
                    gi = kk * _RING + s

                    @pl.when((gi >= _RING) & (gi - _RING < n_grp))
                    def _(s=s):
                        on_my_sc(lambda sp: pltpu.make_async_copy(
                            grp_v.at[s], sp.at[zidx16], sems_r[s]).wait())

                    @pl.when(gi < n_grp)
                    def _(s=s, gi=gi):
                        iota = lax.iota(jnp.int32, _L)
                        jv_raw = wl_v[pl.ds(gi * _L, _L)]
                        valid = (gi * _L + iota) < n_mine
                        jv0 = jnp.broadcast_to(jv_raw[0], (_L,))
                        jv = jnp.where(valid, jv_raw, jv0)
                        idxs = plsc.load_gather(
                            idx_v, [jnp.full((_L,), buf, jnp.int32),
                                    jv - base])
                        bslot = idxs >> 12
                        lane = idxs & 127
                        sv = jnp.full((_L,), s, jnp.int32)
                        for c in range(_D):
                            vals = plsc.load_gather(
                                blocks_v,
                                [bslot, jnp.full((_L,), c, jnp.int32), lane])
                            plsc.store_scatter(
                                grp_v,
                                [sv, iota, jnp.full((_L,), c, jnp.int32)],
                                vals)
                        on_my_sc(lambda sp: pltpu.async_copy(
                            grp_v.at[s], sp.at[jv], sems_r[s]))
                return 0

            n_kk = (n_grp + _RING - 1) // _RING
            lax.fori_loop(0, n_kk, _ext4_body, 0)

            for s in range(_RING):
                @pl.when(s < n_grp)
                def _(s=s):
                    on_my_sc(lambda sp: pltpu.make_async_copy(
                        grp_v.at[s], sp.at[zidx16], sems_r[s]).wait())
            return carry

        lax.fori_loop(0, _B // _CHUNK, _chunk_body, 0)

        plsc.subcore_barrier()

        def _write_block(blk, _):
            for half in range(128 // _SLAB):
                j0 = sid * _JPT + blk * 128 + half * _SLAB
                on_my_sc(lambda sp: pltpu.sync_copy(
                    sp.at[pl.ds(j0, _SLAB), pl.ds(0, _D)],
                    slab_v.at[:, pl.ds(0, _D)]))

                for c in range(_D):
                    for q in range(_SLAB // _L):
                        iota = lax.iota(jnp.int32, _L)
                        v = plsc.load_gather(
                            slab_v,
                            [q * _L + iota, jnp.full((_L,), c, jnp.int32)])
                        asm_v[c, pl.ds(half * _SLAB + q * _L, _L)] = v

            jout = sid * _JPT + blk * 128
            @pl.when(cid == 0)
            def _():
                pltpu.sync_copy(asm_v, out0_hbm.at[:, pl.ds(jout, 128)])
            @pl.when(cid == 1)
            def _():
                pltpu.sync_copy(asm_v, out1_hbm.at[:, pl.ds(jout, 128)])
            return 0

        lax.fori_loop(0, _JPT // 128, _write_block, 0)

    o0, o1 = _emb(x, table.T)
    return (o0 + o1).T


# speedup vs baseline: 1.4841x; 1.2955x over previous
import functools

import jax
import jax.numpy as jnp
from jax import lax
from jax.experimental import pallas as pl
from jax.experimental.pallas import tpu as pltpu
from jax.experimental.pallas import tpu_sc as plsc

_NC = 2
_NS = 16
_NW = _NC * _NS
_CHUNK = 128


def kernel(x, table):
    (B,) = x.shape
    V, D = table.shape
    b_per_w = B // _NW
    n_chunks = b_per_w // _CHUNK

    mesh = plsc.VectorSubcoreMesh(core_axis_name="c", subcore_axis_name="s")

    @functools.partial(
        pl.kernel,
        mesh=mesh,
        out_type=jax.ShapeDtypeStruct((B, D), jnp.float32),
        scratch_types=[
            pltpu.VMEM((n_chunks, _CHUNK), jnp.int32),
            pltpu.VMEM((b_per_w, D), jnp.float32),
            pltpu.SemaphoreType.DMA,
        ],
        compiler_params=pltpu.CompilerParams(use_tc_tiling_on_sc=False),
    )
    def _emb(x_hbm, table_hbm, out_hbm, idx_v, rows_v, sem):
        wid = lax.axis_index("s") * _NC + lax.axis_index("c")
        base = wid * b_per_w
        for j in range(n_chunks):
            pltpu.sync_copy(x_hbm.at[pl.ds(base + j * _CHUNK, _CHUNK)], idx_v.at[j])
        copies = []
        for j in range(n_chunks):
            copies.append(
                pltpu.async_copy(
                    table_hbm.at[idx_v.at[j]],
                    rows_v.at[pl.ds(j * _CHUNK, _CHUNK)],
                    sem,
                )
            )
        for c in copies:
            c.wait()
        pltpu.sync_copy(rows_v, out_hbm.at[pl.ds(base, b_per_w)])

    return _emb(x, table)
